# baseline (speedup 1.0000x reference)
import math

import jax
import jax.numpy as jnp
from jax import lax
from jax.experimental import pallas as pl
from jax.experimental.pallas import tpu as pltpu
from jax.experimental.pallas import tpu_sc as plsc

MODEL_DIM = 128
PAD = 0
SCALE = math.sqrt(float(MODEL_DIM))
NC, NS, L = 2, 16, 16
NW = NC * NS
CHUNK = 256
GP = 128
GPART = CHUNK // GP
NBUF = 3
AHEAD = 2
NGRP = CHUNK // L


def _bcast(vec, lane):
    return lax.gather(
        vec, jnp.full((L, 1), lane, jnp.int32),
        lax.GatherDimensionNumbers(
            offset_dims=(), collapsed_slice_dims=(0,), start_index_map=(0,)),
        (1,), mode=lax.GatherScatterMode.PROMISE_IN_BOUNDS)


def _emb_body(idx_hbm, table_hbm, out_hbm, idx_all, rows, sin, sout):
    wid = lax.axis_index("s") * NC + lax.axis_index("c")
    b_per_w = out_hbm.shape[0] // NW
    base = wid * b_per_w
    nch = b_per_w // CHUNK
    nmain = (nch // NBUF) * NBUF

    pltpu.sync_copy(idx_hbm.at[pl.ds(base, b_per_w)], idx_all)

    def start_gather(g, b):
        for p in range(GPART):
            pltpu.make_async_copy(
                table_hbm.at[idx_all.at[pl.ds(g * CHUNK + p * GP, GP)]],
                rows[b].at[pl.ds(p * GP, GP)], sin[b]).start()

    def wait_gather(b):
        for p in range(GPART):
            pltpu.make_async_copy(
                table_hbm.at[idx_all.at[pl.ds(p * GP, GP)]],
                rows[b].at[pl.ds(p * GP, GP)], sin[b]).wait()

    def start_out(g, b):
        pltpu.make_async_copy(
            rows[b], out_hbm.at[pl.ds(base + g * CHUNK, CHUNK)],
            sout[b]).start()

    def wait_out(b):
        pltpu.make_async_copy(
            rows[b], out_hbm.at[pl.ds(base, CHUNK)], sout[b]).wait()

    def process(g, b):
        wait_gather(b)
        coff = g * CHUNK

        @pl.loop(0, NGRP)
        def _grp(gg):
            iv = idx_all[pl.ds(coff + gg * L, L)]
            cf = jnp.where(iv == PAD, 0.0, SCALE).astype(jnp.float32)
            for lane in range(L):
                bc = _bcast(cf, lane)
                r = gg * L + lane
                for j in range(MODEL_DIM // L):
                    sl = pl.ds(j * L, L)
                    rows[b][r, sl] = rows[b][r, sl] * bc

        start_out(g, b)

        g2 = g + AHEAD
        b2 = (b + AHEAD) % NBUF

        @pl.when(g2 < nch)
        def _ahead():
            @pl.when(g2 >= NBUF)
            def _reuse():
                wait_out(b2)

            start_gather(g2, b2)

    for g in range(AHEAD):
        start_gather(g, g)

    @pl.loop(0, nmain, step=NBUF)
    def _outer(go):
        for b in range(NBUF):
            process(go + b, b)

    for g in range(nmain, nch):
        process(g, g % NBUF)

    for b in range(NBUF):
        wait_out(b)


def kernel(x, table):
    r, c = x.shape
    b = r * c
    idx = x.reshape(b).astype(jnp.int32)
    mesh = plsc.VectorSubcoreMesh(
        core_axis_name="c", subcore_axis_name="s", num_cores=NC, num_subcores=NS
    )
    b_per_w = b // NW
    out = pl.kernel(
        lambda ih, th, oh, ia, *s:
            _emb_body(ih, th, oh, ia, list(s[:NBUF]),
                      list(s[NBUF:2 * NBUF]), list(s[2 * NBUF:])),
        out_type=jax.ShapeDtypeStruct((b, MODEL_DIM), jnp.float32),
        mesh=mesh,
        scratch_types=(
            [pltpu.VMEM((b_per_w,), jnp.int32)]
            + [pltpu.VMEM((CHUNK, MODEL_DIM), jnp.float32)] * NBUF
            + [pltpu.SemaphoreType.DMA] * (2 * NBUF)
        ),
    )(idx, table)
    return out.reshape(r, c, MODEL_DIM)


# speedup vs baseline: 1.0055x; 1.0055x over previous
import math

import jax
import jax.numpy as jnp
from jax import lax
from jax.experimental import pallas as pl
from jax.experimental.pallas import tpu as pltpu
from jax.experimental.pallas import tpu_sc as plsc

MODEL_DIM = 128
PAD = 0
SCALE = math.sqrt(float(MODEL_DIM))
NC, NS, L = 2, 16, 16
NW = NC * NS
CHUNK = 160
GP = 80
GPART = CHUNK // GP
NBUF = 4
AHEAD = 2
NGRP = CHUNK // L


def _bcast(vec, lane):
    return lax.gather(
        vec, jnp.full((L, 1), lane, jnp.int32),
        lax.GatherDimensionNumbers(
            offset_dims=(), collapsed_slice_dims=(0,), start_index_map=(0,)),
        (1,), mode=lax.GatherScatterMode.PROMISE_IN_BOUNDS)


def _emb_body(idx_hbm, table_hbm, out_hbm, idx_all, rows, sin, sout):
    wid = lax.axis_index("s") * NC + lax.axis_index("c")
    b_per_w = out_hbm.shape[0] // NW
    base = wid * b_per_w
    nch = b_per_w // CHUNK
    nmain = (nch // NBUF) * NBUF

    pltpu.sync_copy(idx_hbm.at[pl.ds(base, b_per_w)], idx_all)

    def start_gather(g, b):
        for p in range(GPART):
            pltpu.make_async_copy(
                table_hbm.at[idx_all.at[pl.ds(g * CHUNK + p * GP, GP)]],
                rows[b].at[pl.ds(p * GP, GP)], sin[b]).start()

    def wait_gather(b):
        for p in range(GPART):
            pltpu.make_async_copy(
                table_hbm.at[idx_all.at[pl.ds(p * GP, GP)]],
                rows[b].at[pl.ds(p * GP, GP)], sin[b]).wait()

    def start_out(g, b):
        pltpu.make_async_copy(
            rows[b], out_hbm.at[pl.ds(base + g * CHUNK, CHUNK)],
            sout[b]).start()

    def wait_out(b):
        pltpu.make_async_copy(
            rows[b], out_hbm.at[pl.ds(base, CHUNK)], sout[b]).wait()

    def process(g, b):
        wait_gather(b)
        coff = g * CHUNK

        @pl.loop(0, NGRP)
        def _grp(gg):
            iv = idx_all[pl.ds(coff + gg * L, L)]
            cf = jnp.where(iv == PAD, 0.0, SCALE).astype(jnp.float32)
            for lane in range(L):
                bc = _bcast(cf, lane)
                r = gg * L + lane
                for j in range(MODEL_DIM // L):
                    sl = pl.ds(j * L, L)
                    rows[b][r, sl] = rows[b][r, sl] * bc

        start_out(g, b)

        g2 = g + AHEAD
        b2 = (b + AHEAD) % NBUF

        @pl.when(g2 < nch)
        def _ahead():
            @pl.when(g2 >= NBUF)
            def _reuse():
                wait_out(b2)

            start_gather(g2, b2)

    for g in range(AHEAD):
        start_gather(g, g)

    @pl.loop(0, nmain, step=NBUF)
    def _outer(go):
        for b in range(NBUF):
            process(go + b, b)

    for g in range(nmain, nch):
        process(g, g % NBUF)

    for b in range(NBUF):
        wait_out(b)


def kernel(x, table):
    r, c = x.shape
    b = r * c
    idx = x.reshape(b).astype(jnp.int32)
    mesh = plsc.VectorSubcoreMesh(
        core_axis_name="c", subcore_axis_name="s", num_cores=NC, num_subcores=NS
    )
    b_per_w = b // NW
    out = pl.kernel(
        lambda ih, th, oh, ia, *s:
            _emb_body(ih, th, oh, ia, list(s[:NBUF]),
                      list(s[NBUF:2 * NBUF]), list(s[2 * NBUF:])),
        out_type=jax.ShapeDtypeStruct((b, MODEL_DIM), jnp.float32),
        mesh=mesh,
        scratch_types=(
            [pltpu.VMEM((b_per_w,), jnp.int32)]
            + [pltpu.VMEM((CHUNK, MODEL_DIM), jnp.float32)] * NBUF
            + [pltpu.SemaphoreType.DMA] * (2 * NBUF)
        ),
    )(idx, table)
    return out.reshape(r, c, MODEL_DIM)
